# baseline (device time: 66550 ns/iter reference)
import jax
import jax.numpy as jnp
from jax import lax
from jax.experimental import pallas as pl
from jax.experimental.pallas import tpu as pltpu

N_DEV = 8

G = (0, 1, 3, 2, 4, 5, 7, 6)

AXIS_SLOT_BIT = {"x": 1, "y": 2, "z": 4}
AXIS_DEV_MASK = {"x": 1, "y": 3, "z": 4}

FLOW_ORDERS = (
    ("x", "y", "z"),
    ("y", "z", "x"),
    ("z", "x", "y"),
    ("y", "x", "z"),
)
FLOW_COLS = ((0, 640), (640, 1280), (1280, 1920), (1920, 2048))
N_FLOWS = len(FLOW_ORDERS)
FLOW_SCHED = (0, 2, 1, 3)
LAND_BASE = (0, 4, 6)


def _round_slots(order):
    active = list(range(8))
    rounds = []
    bits = [AXIS_SLOT_BIT[a] for a in order]
    for r, axis in enumerate(order):
        bit = bits[r]
        sent = [j for j in active if j & bit]
        if r + 1 < len(bits):
            nbit = bits[r + 1]
            sent.sort(key=lambda j: 0 if (j ^ bit) & nbit else 1)
        kept = [j ^ bit for j in sent]
        rounds.append((axis, sent, kept))
        active = sorted(kept)
    return rounds


def kernel(x, w_mat):
    m, k_shard = x.shape
    _, n = w_mat.shape
    m_per = m // N_DEV

    flow_rounds = [_round_slots(o) for o in FLOW_ORDERS]

    def body(x_ref, w_ref, out_ref,
             acc0, acc1, acc2, acc3, land0, land1, land2, land3,
             send_sems, recv_sems):
        accs = (acc0, acc1, acc2, acc3)
        lands = (land0, land1, land2, land3)
        my_pos = lax.axis_index("i")

        barrier_sem = pltpu.get_barrier_semaphore()
        for mask in (1, 3, 4):
            pl.semaphore_signal(
                barrier_sem, inc=1,
                device_id=(my_pos ^ mask,),
                device_id_type=pl.DeviceIdType.MESH,
            )
        pl.semaphore_wait(barrier_sem, 3)

        def seed(f, j):
            c = my_pos ^ G[j]
            lo, hi = FLOW_COLS[f]
            xs = x_ref[pl.ds(c * m_per, m_per), :]
            accs[f][j] = jnp.dot(xs, w_ref[:, lo:hi],
                                 preferred_element_type=jnp.float32)

        def start_msg(f, r, i):
            axis, sent, _ = flow_rounds[f][r]
            partner = my_pos ^ AXIS_DEV_MASK[axis]
            rdma = pltpu.make_async_remote_copy(
                src_ref=accs[f].at[sent[i]],
                dst_ref=lands[f].at[LAND_BASE[r] + i],
                send_sem=send_sems.at[f, r, i],
                recv_sem=recv_sems.at[f, r, i],
                device_id=(partner,),
                device_id_type=pl.DeviceIdType.MESH,
            )
            rdma.start()
            return rdma

        def start_round(f, r):
            return [start_msg(f, r, i) for i in range(len(flow_rounds[f][r][1]))]

        def fold(f, r, rdmas, i):
            _, _, kept = flow_rounds[f][r]
            rdmas[i].wait()
            folded = accs[f][kept[i]] + lands[f][LAND_BASE[r] + i]
            if r == 2:
                lo, hi = FLOW_COLS[f]
                out_ref[:, lo:hi] = folded
            else:
                accs[f][kept[i]] = folded

        inflight = [[] for _ in range(N_FLOWS)]
        for i in range(4):
            for f in FLOW_SCHED:
                seed(f, flow_rounds[f][0][1][i])
                inflight[f].append(start_msg(f, 0, i))
        for f in FLOW_SCHED:
            for j in flow_rounds[f][0][2]:
                seed(f, j)

        for r in (1, 2):
            prev = list(inflight)
            for f in FLOW_SCHED:
                n_need = len(flow_rounds[f][r][1])
                for i in range(n_need):
                    fold(f, r - 1, prev[f], i)
                inflight[f] = start_round(f, r)
            for f in FLOW_SCHED:
                n_need = len(flow_rounds[f][r][1])
                for i in range(n_need, len(prev[f])):
                    fold(f, r - 1, prev[f], i)
        for f in FLOW_SCHED:
            fold(f, 2, inflight[f], 0)

    widths = [hi - lo for lo, hi in FLOW_COLS]
    return pl.pallas_call(
        body,
        out_shape=jax.ShapeDtypeStruct((m_per, n), jnp.float32),
        in_specs=[
            pl.BlockSpec(memory_space=pltpu.VMEM),
            pl.BlockSpec(memory_space=pltpu.VMEM),
        ],
        out_specs=pl.BlockSpec(memory_space=pltpu.VMEM),
        scratch_shapes=(
            [pltpu.VMEM((N_DEV, m_per, w), jnp.float32) for w in widths]
            + [pltpu.VMEM((7, m_per, w), jnp.float32) for w in widths]
            + [
                pltpu.SemaphoreType.DMA((N_FLOWS, 3, 4)),
                pltpu.SemaphoreType.DMA((N_FLOWS, 3, 4)),
            ]
        ),
        compiler_params=pltpu.CompilerParams(collective_id=0),
    )(x, w_mat)
